# baseline (speedup 1.0000x reference)
import jax
import jax.numpy as jnp
import numpy as np
from jax.experimental import pallas as pl
from jax.experimental.pallas import tpu as pltpu

B = 64
NB = 8
F32 = jnp.float32


def _lrelu(v):
    return jnp.where(v > 0, v, 0.01 * v)


def _mmd(w, x):
    return jax.lax.dot_general(w, x, (((1,), (0,)), ((), ())),
                               preferred_element_type=F32,
                               precision=jax.lax.Precision.DEFAULT)


def _encvq_one(nb, xph_ref, w1_ref, w2_ref, w3_ref, prew_ref, preb_ref,
               emb_ref, embt_ref, e2_ref, enc_out_ref, h1_ref, h2_ref):
    rows = []
    for k in range(16):
        segs = []
        for r in range(4):
            jj = 2 * r + k
            segs.append(xph_ref[nb, jj % 8: jj % 8 + 1, jj // 8: jj // 8 + 512])
        rows.append(jnp.concatenate(segs, axis=1))
    X1 = jnp.concatenate(rows, axis=0)
    h1 = _lrelu(_mmd(w1_ref[...], X1))
    for r in range(4):
        h1_ref[nb, 64 * r: 64 * (r + 1), 1:513] = h1[:, 512 * r: 512 * (r + 1)]

    Xe = jnp.concatenate([h1_ref[nb, 64:256, 0:512],
                          h1_ref[nb, 0:256, 1:513],
                          h1_ref[nb, 0:64, 2:514]], axis=0)
    Xo = jnp.concatenate([h1_ref[nb, 192:256, 0:512],
                          h1_ref[nb, 0:256, 1:513],
                          h1_ref[nb, 0:192, 2:514]], axis=0)
    h2 = _lrelu(_mmd(w2_ref[...], jnp.concatenate([Xe, Xo], axis=1)))
    h2_ref[nb, 0:128, 0:512] = h2[:, 0:512]
    h2_ref[nb, 128:256, 1:513] = h2[:, 512:1024]

    X3 = jnp.concatenate([
        h2_ref[nb, 128:256, 0:512],
        h2_ref[nb, 0:128, 0:512],
        h2_ref[nb, 128:256, 1:513],
        h2_ref[nb, 0:128, 1:513],
    ], axis=0)
    h3 = _lrelu(_mmd(w3_ref[...], X3))
    z = _mmd(prew_ref[...], h3) + preb_ref[...]

    scores = _mmd(emb_ref[...], z)
    zsq = jnp.sum(z * z, axis=0, keepdims=True)
    dist = (zsq + e2_ref[...]) - 2.0 * scores
    minv = jnp.min(dist, axis=0, keepdims=True)
    iota = jax.lax.broadcasted_iota(jnp.int32, (1024, 512), 0)
    sel = jnp.where(dist == minv, iota, jnp.int32(2 ** 30))
    idx = jnp.min(sel, axis=0, keepdims=True)
    onehot = (iota == idx).astype(F32)
    q = _mmd(embt_ref[...], onehot)
    enc_out_ref[nb] = q
    c_part = jnp.sum(onehot, axis=1, keepdims=True)
    s_part = jnp.sum((q - z) ** 2).reshape(1, 1)
    return q, c_part, s_part


def _encvq_body(xph_ref, w1_ref, w2_ref, w3_ref, prew_ref, preb_ref,
                emb_ref, embt_ref, e2_ref, w0_ref, b0_ref, wd1_ref, wd2_ref,
                wd3_ref,
                enc_out_ref, counts_ref, sse_ref, dec_out_ref,
                h1_ref, h2_ref, ep_ref, d0_ref, d1_ref, d2_ref):
    b = pl.program_id(0)

    @pl.when(b == 0)
    def _zero_edges():
        h1_ref[:, :, 0:1] = jnp.zeros((NB, 256, 1), F32)
        h1_ref[:, :, 513:514] = jnp.zeros((NB, 256, 1), F32)
        h2_ref[:, :, 0:1] = jnp.zeros((NB, 256, 1), F32)
        h2_ref[:, :, 512:514] = jnp.zeros((NB, 256, 2), F32)
        for ref, c in ((ep_ref, 64), (d0_ref, 128), (d1_ref, 256),
                       (d2_ref, 256)):
            ref[:, :, 0:1] = jnp.zeros((NB, c, 1), F32)
            ref[:, :, 513:514] = jnp.zeros((NB, c, 1), F32)

    c_tot = jnp.zeros((1024, 1), F32)
    s_tot = jnp.zeros((1, 1), F32)
    for nb in range(NB):
        q, c_p, s_p = _encvq_one(nb, xph_ref, w1_ref, w2_ref, w3_ref, prew_ref,
                                 preb_ref, emb_ref, embt_ref, e2_ref,
                                 enc_out_ref, h1_ref, h2_ref)
        _dec_one_q(nb, q, w0_ref, b0_ref, wd1_ref, wd2_ref, wd3_ref,
                   dec_out_ref, ep_ref, d0_ref, d1_ref, d2_ref)
        c_tot = c_tot + c_p
        s_tot = s_tot + s_p

    @pl.when(b == 0)
    def _init():
        counts_ref[...] = c_tot
        sse_ref[...] = s_tot

    @pl.when(b > 0)
    def _accum():
        counts_ref[...] = counts_ref[...] + c_tot
        sse_ref[...] = sse_ref[...] + s_tot


_REG_STEPS = 8


def _reg_body(enc_ref, w1_ref, b1_ref, w2t_ref, b2_ref, counts_ref, sse_ref,
              freq_ref, perp_ref, loss_ref, acc_ref):
    g = pl.program_id(0)
    part = jnp.zeros((B, 256), F32)
    for j in range(8):
        part = part + jax.lax.dot_general(
            enc_ref[:, j, :], w1_ref[:, 512 * j: 512 * (j + 1)],
            (((1,), (1,)), ((), ())), preferred_element_type=F32,
            precision=jax.lax.Precision.DEFAULT)

    @pl.when(g == 0)
    def _init():
        acc_ref[...] = part

    @pl.when(g > 0)
    def _accum():
        acc_ref[...] = acc_ref[...] + part

    @pl.when(g == _REG_STEPS - 1)
    def _final():
        h = acc_ref[...] + b1_ref[...]
        f = jax.lax.dot_general(h, w2t_ref[...], (((1,), (0,)), ((), ())),
                                preferred_element_type=F32,
                                precision=jax.lax.Precision.DEFAULT) + b2_ref[...]
        freq_ref[...] = jax.nn.sigmoid(f)
        avg = counts_ref[...] * (1.0 / 32768.0)
        perp_ref[...] = jnp.exp(
            -jnp.sum(avg * jnp.log(avg + 1e-10))).reshape(1, 1)
        loss_ref[...] = sse_ref[...] * (1.25 / 2097152.0)


def _dec_one_q(nb, q, w0_ref, b0_ref, wd1_ref, wd2_ref, wd3_ref,
               out_ref, ep_ref, d0_ref, d1_ref, d2_ref):
    ep_ref[nb, :, 1:513] = q
    X0 = jnp.concatenate([ep_ref[nb, :, 0:512], ep_ref[nb, :, 1:513],
                          ep_ref[nb, :, 2:514]], axis=0)
    d0_ref[nb, :, 1:513] = _mmd(w0_ref[...], X0) + b0_ref[...]

    X1 = jnp.concatenate([d0_ref[nb, :, 0:512], d0_ref[nb, :, 1:513],
                          d0_ref[nb, :, 2:514]], axis=0)
    d1_ref[nb, :, 1:513] = _lrelu(_mmd(wd1_ref[...], X1))

    X2 = jnp.concatenate([d1_ref[nb, :, 0:512], d1_ref[nb, :, 1:513],
                          d1_ref[nb, :, 2:514]], axis=0)
    d2_ref[nb, :, 1:513] = _lrelu(_mmd(wd2_ref[...], X2))

    X3 = jnp.concatenate([d2_ref[nb, :, 0:512], d2_ref[nb, :, 1:513],
                          d2_ref[nb, :, 2:514]], axis=0)
    out_ref[nb] = jax.nn.sigmoid(_mmd(wd3_ref[...], X3))


def _bspec(shape, mode):
    if mode == "lead":
        return pl.BlockSpec(shape, lambda b: (b,) + (0,) * (len(shape) - 1))
    return pl.BlockSpec(shape, lambda b: (0,) * len(shape))


def kernel(x, enc_w1, enc_w2, enc_w3, pre_w, pre_b, emb, reg_w1, reg_b1,
           reg_w2, reg_b2, dec_w0, dec_b0, dect_w1, dect_w2, dect_w3):
    xp = jnp.pad(x[:, 0, :], ((0, 0), (7, 25)))
    xph = jnp.transpose(xp.reshape(B, 516, 8), (0, 2, 1))
    w1 = enc_w1[:, 0, :]
    w2 = jnp.transpose(enc_w2, (0, 2, 1)).reshape(128, 512)
    w3 = jnp.transpose(enc_w3, (0, 2, 1)).reshape(128, 512)
    prew = pre_w[:, :, 0]
    preb = pre_b.reshape(64, 1)
    embt = emb.T
    e2 = jnp.sum(emb ** 2, axis=1).reshape(1024, 1)

    w0 = jnp.transpose(dec_w0, (0, 2, 1)).reshape(128, 192)
    b0 = dec_b0.reshape(128, 1)
    wtd1 = jnp.transpose(jnp.flip(dect_w1, 2), (1, 0, 2))
    wtd1p = jnp.concatenate([wtd1, jnp.zeros((128, 128, 1), F32)], axis=2)
    J1 = np.array([[0, 2, 4], [4, 1, 3]])
    wd1 = jnp.transpose(wtd1p[:, :, J1], (2, 0, 3, 1)).reshape(256, 384)
    wtd2 = jnp.transpose(jnp.flip(dect_w2, 2), (1, 0, 2))
    wtd2p = jnp.concatenate([wtd2, jnp.zeros((64, 128, 1), F32)], axis=2)
    J2 = np.array([[0, 2, 4, 6, 8, 8],
                   [8, 1, 3, 5, 7, 8],
                   [8, 0, 2, 4, 6, 8],
                   [8, 8, 1, 3, 5, 7]])
    wd2 = jnp.transpose(wtd2p[:, :, J2], (2, 0, 3, 1)).reshape(256, 768)
    wtd3 = jnp.transpose(jnp.flip(dect_w3, 2), (1, 0, 2))[0]
    wtd3p = jnp.concatenate([wtd3, jnp.zeros((64, 1), F32)], axis=1)
    J3 = np.full((8, 12), 16)
    for t in range(4):
        for r in range(2):
            for o in range(3):
                for qp in range(4):
                    c = qp + 4 * (o - 1) - t
                    ok = (-4 <= c <= 3) if r == 0 else (-3 <= c <= 4)
                    if ok:
                        J3[2 * t + r, 4 * o + qp] = (2 * c + 8) if r == 0 else (2 * c + 7)
    wd3 = jnp.transpose(wtd3p[:, J3], (1, 2, 0)).reshape(8, 768)

    _ENCDEC_CALL = lambda *a: pl.pallas_call(
        _encvq_body, grid=(B // NB,),
        in_specs=[_bspec((NB, 8, 516), "lead"), _bspec((64, 16), None),
                  _bspec((128, 512), None), _bspec((128, 512), None),
                  _bspec((64, 128), None), _bspec((64, 1), None),
                  _bspec((1024, 64), None), _bspec((64, 1024), None),
                  _bspec((1024, 1), None), _bspec((128, 192), None),
                  _bspec((128, 1), None), _bspec((256, 384), None),
                  _bspec((256, 768), None), _bspec((8, 768), None)],
        out_specs=[_bspec((NB, 64, 512), "lead"), _bspec((1024, 1), None),
                   _bspec((1, 1), None), _bspec((NB, 8, 512), "lead")],
        out_shape=[jax.ShapeDtypeStruct((B, 64, 512), F32),
                   jax.ShapeDtypeStruct((1024, 1), F32),
                   jax.ShapeDtypeStruct((1, 1), F32),
                   jax.ShapeDtypeStruct((B, 8, 512), F32)],
        scratch_shapes=[pltpu.VMEM((NB, 256, 514), F32),
                        pltpu.VMEM((NB, 256, 514), F32),
                        pltpu.VMEM((NB, 64, 514), F32),
                        pltpu.VMEM((NB, 128, 514), F32),
                        pltpu.VMEM((NB, 256, 514), F32),
                        pltpu.VMEM((NB, 256, 514), F32)],
    )(*a, w0, b0, wd1, wd2, wd3)

    encoded, counts, sse, dec8 = _ENCDEC_CALL(
        xph, w1, w2, w3, prew, preb, emb, embt, e2)

    freq, perp, loss = pl.pallas_call(
        _reg_body, grid=(_REG_STEPS,),
        in_specs=[pl.BlockSpec((B, 8, 512), lambda g: (0, g, 0)),
                  pl.BlockSpec((256, 4096), lambda g: (0, g)),
                  _bspec((1, 256), None), _bspec((256, 6), None),
                  _bspec((1, 6), None), _bspec((1024, 1), None),
                  _bspec((1, 1), None)],
        out_specs=[_bspec((B, 6), None), _bspec((1, 1), None),
                   _bspec((1, 1), None)],
        out_shape=[jax.ShapeDtypeStruct((B, 6), F32),
                   jax.ShapeDtypeStruct((1, 1), F32),
                   jax.ShapeDtypeStruct((1, 1), F32)],
        scratch_shapes=[pltpu.VMEM((B, 256), F32)],
    )(encoded, reg_w1, reg_b1.reshape(1, 256), reg_w2.T,
      reg_b2.reshape(1, 6), counts, sse)

    decoded = jnp.transpose(dec8, (0, 2, 1)).reshape(B, 1, 4096)

    return encoded, perp.reshape(()), loss.reshape(()), freq, decoded


# speedup vs baseline: 1.0422x; 1.0422x over previous
import jax
import jax.numpy as jnp
import numpy as np
from jax.experimental import pallas as pl
from jax.experimental.pallas import tpu as pltpu

B = 64
NB = 8
F32 = jnp.float32


def _lrelu(v):
    return jnp.where(v > 0, v, 0.01 * v)


def _mmd(w, x):
    return jax.lax.dot_general(w, x, (((1,), (0,)), ((), ())),
                               preferred_element_type=F32,
                               precision=jax.lax.Precision.DEFAULT)


def _encvq_one(nb, xph_ref, w1_ref, w2_ref, w3_ref, prew_ref, preb_ref,
               emb_ref, embt_ref, e2_ref, enc_out_ref, h1_ref, h2_ref):
    rows = []
    for k in range(16):
        segs = []
        for r in range(4):
            jj = 2 * r + k
            segs.append(xph_ref[nb, jj % 8: jj % 8 + 1, jj // 8: jj // 8 + 512])
        rows.append(jnp.concatenate(segs, axis=1))
    X1 = jnp.concatenate(rows, axis=0)
    h1 = _lrelu(_mmd(w1_ref[...], X1))
    for r in range(4):
        h1_ref[nb, 64 * r: 64 * (r + 1), 1:513] = h1[:, 512 * r: 512 * (r + 1)]

    Xe = jnp.concatenate([h1_ref[nb, 64:256, 0:512],
                          h1_ref[nb, 0:256, 1:513],
                          h1_ref[nb, 0:64, 2:514]], axis=0)
    Xo = jnp.concatenate([h1_ref[nb, 192:256, 0:512],
                          h1_ref[nb, 0:256, 1:513],
                          h1_ref[nb, 0:192, 2:514]], axis=0)
    h2 = _lrelu(_mmd(w2_ref[...], jnp.concatenate([Xe, Xo], axis=1)))
    h2_ref[nb, 0:128, 0:512] = h2[:, 0:512]
    h2_ref[nb, 128:256, 1:513] = h2[:, 512:1024]

    X3 = jnp.concatenate([
        h2_ref[nb, 128:256, 0:512],
        h2_ref[nb, 0:128, 0:512],
        h2_ref[nb, 128:256, 1:513],
        h2_ref[nb, 0:128, 1:513],
    ], axis=0)
    h3 = _lrelu(_mmd(w3_ref[...], X3))
    z = _mmd(prew_ref[...], h3) + preb_ref[...]

    scores = _mmd(emb_ref[...], z)
    zsq = jnp.sum(z * z, axis=0, keepdims=True)
    dist = (zsq + e2_ref[...]) - 2.0 * scores
    minv = jnp.min(dist, axis=0, keepdims=True)
    iota = jax.lax.broadcasted_iota(jnp.int32, (1024, 512), 0)
    sel = jnp.where(dist == minv, iota, jnp.int32(2 ** 30))
    idx = jnp.min(sel, axis=0, keepdims=True)
    onehot = (iota == idx).astype(F32)
    q = _mmd(embt_ref[...], onehot)
    enc_out_ref[nb] = q
    c_part = jnp.sum(onehot, axis=1, keepdims=True)
    s_part = jnp.sum((q - z) ** 2).reshape(1, 1)
    return c_part, s_part


def _encvq_body(xph_ref, w1_ref, w2_ref, w3_ref, prew_ref, preb_ref,
                emb_ref, embt_ref, e2_ref,
                enc_out_ref, counts_ref, sse_ref, h1_ref, h2_ref):
    b = pl.program_id(0)

    @pl.when(b == 0)
    def _zero_edges():
        h1_ref[:, :, 0:1] = jnp.zeros((NB, 256, 1), F32)
        h1_ref[:, :, 513:514] = jnp.zeros((NB, 256, 1), F32)
        h2_ref[:, :, 0:1] = jnp.zeros((NB, 256, 1), F32)
        h2_ref[:, :, 512:514] = jnp.zeros((NB, 256, 2), F32)

    c_tot = jnp.zeros((1024, 1), F32)
    s_tot = jnp.zeros((1, 1), F32)
    for nb in range(NB):
        c_p, s_p = _encvq_one(nb, xph_ref, w1_ref, w2_ref, w3_ref, prew_ref,
                              preb_ref, emb_ref, embt_ref, e2_ref,
                              enc_out_ref, h1_ref, h2_ref)
        c_tot = c_tot + c_p
        s_tot = s_tot + s_p

    @pl.when(b == 0)
    def _init():
        counts_ref[...] = c_tot
        sse_ref[...] = s_tot

    @pl.when(b > 0)
    def _accum():
        counts_ref[...] = counts_ref[...] + c_tot
        sse_ref[...] = sse_ref[...] + s_tot


_REG_STEPS = 8


def _reg_body(enc_ref, w1_ref, b1_ref, w2t_ref, b2_ref, counts_ref, sse_ref,
              freq_ref, perp_ref, loss_ref, acc_ref):
    g = pl.program_id(0)
    part = jnp.zeros((B, 256), F32)
    for j in range(8):
        part = part + jax.lax.dot_general(
            enc_ref[:, j, :], w1_ref[:, 512 * j: 512 * (j + 1)],
            (((1,), (1,)), ((), ())), preferred_element_type=F32,
            precision=jax.lax.Precision.DEFAULT)

    @pl.when(g == 0)
    def _init():
        acc_ref[...] = part

    @pl.when(g > 0)
    def _accum():
        acc_ref[...] = acc_ref[...] + part

    @pl.when(g == _REG_STEPS - 1)
    def _final():
        h = acc_ref[...] + b1_ref[...]
        f = jax.lax.dot_general(h, w2t_ref[...], (((1,), (0,)), ((), ())),
                                preferred_element_type=F32,
                                precision=jax.lax.Precision.DEFAULT) + b2_ref[...]
        freq_ref[...] = jax.nn.sigmoid(f)
        avg = counts_ref[...] * (1.0 / 32768.0)
        perp_ref[...] = jnp.exp(
            -jnp.sum(avg * jnp.log(avg + 1e-10))).reshape(1, 1)
        loss_ref[...] = sse_ref[...] * (1.25 / 2097152.0)


def _dec_one(nb, enc_ref, w0_ref, b0_ref, wd1_ref, wd2_ref, wd3_ref,
             out_ref, ep_ref, d0_ref, d1_ref, d2_ref):
    ep_ref[nb, :, 1:513] = enc_ref[nb]
    X0 = jnp.concatenate([ep_ref[nb, :, 0:512], ep_ref[nb, :, 1:513],
                          ep_ref[nb, :, 2:514]], axis=0)
    d0_ref[nb, :, 1:513] = _mmd(w0_ref[...], X0) + b0_ref[...]

    X1 = jnp.concatenate([d0_ref[nb, :, 0:512], d0_ref[nb, :, 1:513],
                          d0_ref[nb, :, 2:514]], axis=0)
    d1_ref[nb, :, 1:513] = _lrelu(_mmd(wd1_ref[...], X1))

    X2 = jnp.concatenate([d1_ref[nb, :, 0:512], d1_ref[nb, :, 1:513],
                          d1_ref[nb, :, 2:514]], axis=0)
    d2_ref[nb, :, 1:513] = _lrelu(_mmd(wd2_ref[...], X2))

    X3 = jnp.concatenate([d2_ref[nb, :, 0:512], d2_ref[nb, :, 1:513],
                          d2_ref[nb, :, 2:514]], axis=0)
    out_ref[nb] = jax.nn.sigmoid(_mmd(wd3_ref[...], X3))


def _dec_body(enc_ref, w0_ref, b0_ref, wd1_ref, wd2_ref, wd3_ref,
              out_ref, ep_ref, d0_ref, d1_ref, d2_ref):
    b = pl.program_id(0)

    @pl.when(b == 0)
    def _zero_edges():
        for ref, c in ((ep_ref, 64), (d0_ref, 128), (d1_ref, 256),
                       (d2_ref, 256)):
            ref[:, :, 0:1] = jnp.zeros((NB, c, 1), F32)
            ref[:, :, 513:514] = jnp.zeros((NB, c, 1), F32)

    for nb in range(NB):
        _dec_one(nb, enc_ref, w0_ref, b0_ref, wd1_ref, wd2_ref, wd3_ref,
                 out_ref, ep_ref, d0_ref, d1_ref, d2_ref)


def _bspec(shape, mode):
    if mode == "lead":
        return pl.BlockSpec(shape, lambda b: (b,) + (0,) * (len(shape) - 1))
    return pl.BlockSpec(shape, lambda b: (0,) * len(shape))


def kernel(x, enc_w1, enc_w2, enc_w3, pre_w, pre_b, emb, reg_w1, reg_b1,
           reg_w2, reg_b2, dec_w0, dec_b0, dect_w1, dect_w2, dect_w3):
    xp = jnp.pad(x[:, 0, :], ((0, 0), (7, 25)))
    xph = jnp.transpose(xp.reshape(B, 516, 8), (0, 2, 1))
    w1 = enc_w1[:, 0, :]
    w2 = jnp.transpose(enc_w2, (0, 2, 1)).reshape(128, 512)
    w3 = jnp.transpose(enc_w3, (0, 2, 1)).reshape(128, 512)
    prew = pre_w[:, :, 0]
    preb = pre_b.reshape(64, 1)
    embt = emb.T
    e2 = jnp.sum(emb ** 2, axis=1).reshape(1024, 1)
    encoded, counts, sse = pl.pallas_call(
        _encvq_body, grid=(B // NB,),
        in_specs=[_bspec((NB, 8, 516), "lead"), _bspec((64, 16), None),
                  _bspec((128, 512), None), _bspec((128, 512), None),
                  _bspec((64, 128), None), _bspec((64, 1), None),
                  _bspec((1024, 64), None), _bspec((64, 1024), None),
                  _bspec((1024, 1), None)],
        out_specs=[_bspec((NB, 64, 512), "lead"), _bspec((1024, 1), None),
                   _bspec((1, 1), None)],
        out_shape=[jax.ShapeDtypeStruct((B, 64, 512), F32),
                   jax.ShapeDtypeStruct((1024, 1), F32),
                   jax.ShapeDtypeStruct((1, 1), F32)],
        scratch_shapes=[pltpu.VMEM((NB, 256, 514), F32),
                        pltpu.VMEM((NB, 256, 514), F32)],
    )(xph, w1, w2, w3, prew, preb, emb, embt, e2)

    freq, perp, loss = pl.pallas_call(
        _reg_body, grid=(_REG_STEPS,),
        in_specs=[pl.BlockSpec((B, 8, 512), lambda g: (0, g, 0)),
                  pl.BlockSpec((256, 4096), lambda g: (0, g)),
                  _bspec((1, 256), None), _bspec((256, 6), None),
                  _bspec((1, 6), None), _bspec((1024, 1), None),
                  _bspec((1, 1), None)],
        out_specs=[_bspec((B, 6), None), _bspec((1, 1), None),
                   _bspec((1, 1), None)],
        out_shape=[jax.ShapeDtypeStruct((B, 6), F32),
                   jax.ShapeDtypeStruct((1, 1), F32),
                   jax.ShapeDtypeStruct((1, 1), F32)],
        scratch_shapes=[pltpu.VMEM((B, 256), F32)],
    )(encoded, reg_w1, reg_b1.reshape(1, 256), reg_w2.T,
      reg_b2.reshape(1, 6), counts, sse)

    w0 = jnp.transpose(dec_w0, (0, 2, 1)).reshape(128, 192)
    b0 = dec_b0.reshape(128, 1)
    wtd1 = jnp.transpose(jnp.flip(dect_w1, 2), (1, 0, 2))
    wtd1p = jnp.concatenate([wtd1, jnp.zeros((128, 128, 1), F32)], axis=2)
    J1 = np.array([[0, 2, 4], [4, 1, 3]])
    wd1 = jnp.transpose(wtd1p[:, :, J1], (2, 0, 3, 1)).reshape(256, 384)
    wtd2 = jnp.transpose(jnp.flip(dect_w2, 2), (1, 0, 2))
    wtd2p = jnp.concatenate([wtd2, jnp.zeros((64, 128, 1), F32)], axis=2)
    J2 = np.array([[0, 2, 4, 6, 8, 8],
                   [8, 1, 3, 5, 7, 8],
                   [8, 0, 2, 4, 6, 8],
                   [8, 8, 1, 3, 5, 7]])
    wd2 = jnp.transpose(wtd2p[:, :, J2], (2, 0, 3, 1)).reshape(256, 768)
    wtd3 = jnp.transpose(jnp.flip(dect_w3, 2), (1, 0, 2))[0]
    wtd3p = jnp.concatenate([wtd3, jnp.zeros((64, 1), F32)], axis=1)
    J3 = np.full((8, 12), 16)
    for t in range(4):
        for r in range(2):
            for o in range(3):
                for qp in range(4):
                    c = qp + 4 * (o - 1) - t
                    ok = (-4 <= c <= 3) if r == 0 else (-3 <= c <= 4)
                    if ok:
                        J3[2 * t + r, 4 * o + qp] = (2 * c + 8) if r == 0 else (2 * c + 7)
    wd3 = jnp.transpose(wtd3p[:, J3], (1, 2, 0)).reshape(8, 768)

    dec8 = pl.pallas_call(
        _dec_body, grid=(B // NB,),
        in_specs=[_bspec((NB, 64, 512), "lead"), _bspec((128, 192), None),
                  _bspec((128, 1), None), _bspec((256, 384), None),
                  _bspec((256, 768), None), _bspec((8, 768), None)],
        out_specs=_bspec((NB, 8, 512), "lead"),
        out_shape=jax.ShapeDtypeStruct((B, 8, 512), F32),
        scratch_shapes=[pltpu.VMEM((NB, 64, 514), F32),
                        pltpu.VMEM((NB, 128, 514), F32),
                        pltpu.VMEM((NB, 256, 514), F32),
                        pltpu.VMEM((NB, 256, 514), F32)],
    )(encoded, w0, b0, wd1, wd2, wd3)
    decoded = jnp.transpose(dec8, (0, 2, 1)).reshape(B, 1, 4096)

    return encoded, perp.reshape(()), loss.reshape(()), freq, decoded
